# baseline (speedup 1.0000x reference)
import functools

import jax
import jax.numpy as jnp
from jax import lax
from jax.experimental import pallas as pl
from jax.experimental.pallas import tpu as pltpu
from jax.experimental.pallas import tpu_sc as plsc

_N_TARGET = 100000
_N_SOURCE = 10000
_N_SYN = 6400000

_NC = 2
_NS = 16
_NW = _NC * _NS
_PAD = 102400
_SYN_PER_TILE = _N_SYN // _NW
_CHUNK = 2000
_NCHUNKS = _SYN_PER_TILE // _CHUNK
_P2_COLS = _PAD // _NW

_mesh = plsc.VectorSubcoreMesh(core_axis_name="c", subcore_axis_name="s")
_params = pltpu.CompilerParams(needs_layout_passes=False)


@functools.partial(
    pl.kernel,
    out_type=jax.ShapeDtypeStruct((_NW, _PAD), jnp.float32),
    mesh=_mesh,
    compiler_params=_params,
    scratch_types=[
        pltpu.VMEM((_PAD,), jnp.float32),
        pltpu.VMEM((_N_SOURCE,), jnp.float32),
        pltpu.VMEM((_CHUNK,), jnp.int32),
        pltpu.VMEM((_CHUNK,), jnp.int32),
        pltpu.VMEM((_CHUNK,), jnp.int32),
        pltpu.VMEM((_CHUNK,), jnp.int32),
        pltpu.VMEM((_CHUNK,), jnp.float32),
        pltpu.VMEM((_CHUNK,), jnp.float32),
        pltpu.VMEM((_CHUNK,), jnp.float32),
        pltpu.VMEM((_CHUNK,), jnp.float32),
        pltpu.SemaphoreType.DMA,
        pltpu.SemaphoreType.DMA,
    ],
)
def _phase1(post_hbm, pre_hbm, w_hbm, wf_hbm, inp_hbm, part_hbm,
            acc, inp_v, post_a, post_b, pre_a, pre_b, w_a, w_b, wf_a, wf_b,
            sem_a, sem_b):
    cid = lax.axis_index("c")
    sid = lax.axis_index("s")
    wid = sid * _NC + cid
    tile_base = wid * _SYN_PER_TILE

    post_bufs = (post_a, post_b)
    pre_bufs = (pre_a, pre_b)
    w_bufs = (w_a, w_b)
    wf_bufs = (wf_a, wf_b)
    sems = (sem_a, sem_b)

    zeros16 = jnp.zeros((16,), jnp.float32)

    def zbody(i, _):
        acc[pl.ds(i * 16, 16)] = zeros16
        return 0

    lax.fori_loop(0, _PAD // 16, zbody, 0)

    pltpu.sync_copy(inp_hbm, inp_v)

    def issue_loads(b, chunk):
        syn0 = tile_base + chunk * _CHUNK
        sl = pl.ds(syn0, _CHUNK)
        pltpu.async_copy(post_hbm.at[sl], post_bufs[b], sems[b])
        pltpu.async_copy(pre_hbm.at[sl], pre_bufs[b], sems[b])
        pltpu.async_copy(w_hbm.at[sl], w_bufs[b], sems[b])
        pltpu.async_copy(wf_hbm.at[sl], wf_bufs[b], sems[b])

    def wait_loads(b):
        sl = pl.ds(0, _CHUNK)
        pltpu.make_async_copy(post_hbm.at[sl], post_bufs[b], sems[b]).wait()
        pltpu.make_async_copy(pre_hbm.at[sl], pre_bufs[b], sems[b]).wait()
        pltpu.make_async_copy(w_hbm.at[sl], w_bufs[b], sems[b]).wait()
        pltpu.make_async_copy(wf_hbm.at[sl], wf_bufs[b], sems[b]).wait()

    issue_loads(0, 0)
    issue_loads(1, 1)

    def compute_chunk(b):
        post_s, pre_s = post_bufs[b], pre_bufs[b]
        w_s, wf_s = w_bufs[b], wf_bufs[b]

        def cbody(i, _):
            for u in range(5):
                sl = pl.ds(i * 80 + u * 16, 16)
                post_v = post_s[sl]
                pre_v = pre_s[sl]
                act = plsc.load_gather(inp_v, [pre_v])
                val = jnp.where(act > 0.0, w_s[sl] * wf_s[sl], 0.0)
                plsc.addupdate_scatter(acc, [post_v], val)
            return 0

        lax.fori_loop(0, _CHUNK // 80, cbody, 0)

    def tbody(t, _):
        for b in range(2):
            j = t * 2 + b
            wait_loads(b)
            compute_chunk(b)

            @pl.when(j + 2 < _NCHUNKS)
            def _():
                issue_loads(b, j + 2)
        return 0

    lax.fori_loop(0, _NCHUNKS // 2, tbody, 0)

    pltpu.sync_copy(acc, part_hbm.at[wid])


@functools.partial(
    pl.kernel,
    out_type=jax.ShapeDtypeStruct((_PAD,), jnp.float32),
    mesh=_mesh,
    compiler_params=_params,
    scratch_types=[
        pltpu.VMEM((_NW, _P2_COLS), jnp.float32),
        pltpu.VMEM((_P2_COLS,), jnp.float32),
    ],
)
def _phase2(part_hbm, out_hbm, buf, outv):
    cid = lax.axis_index("c")
    sid = lax.axis_index("s")
    wid = sid * _NC + cid
    off = wid * _P2_COLS

    pltpu.sync_copy(part_hbm.at[:, pl.ds(off, _P2_COLS)], buf)

    def rbody(i, _):
        s = pl.ds(i * 16, 16)
        v = buf[0, s]
        for k in range(1, _NW):
            v = v + buf[k, s]
        outv[s] = v
        return 0

    lax.fori_loop(0, _P2_COLS // 16, rbody, 0)

    pltpu.sync_copy(outv, out_hbm.at[pl.ds(off, _P2_COLS)])


def kernel(inputs_t, indices, weights, weights_factors):
    post = indices[:, 0]
    pre = indices[:, 1]
    inp_flat = jnp.reshape(inputs_t, (-1,))
    part = _phase1(post, pre, weights, weights_factors, inp_flat)
    out = _phase2(part)
    return out[: _N_TARGET].reshape(inputs_t.shape[0], -1)


# speedup vs baseline: 1.2613x; 1.2613x over previous
import functools

import jax
import jax.numpy as jnp
from jax import lax
from jax.experimental import pallas as pl
from jax.experimental.pallas import tpu as pltpu
from jax.experimental.pallas import tpu_sc as plsc

_N_TARGET = 100000
_N_SOURCE = 10000
_N_SYN = 6400000

_NC = 2
_NS = 16
_NW = _NC * _NS
_PAD = 102400
_SYN_PER_TILE = _N_SYN // _NW
_CHUNK = 2000
_NCHUNKS = _SYN_PER_TILE // _CHUNK
_P2_COLS = _PAD // _NW

_mesh = plsc.VectorSubcoreMesh(core_axis_name="c", subcore_axis_name="s")
_params = pltpu.CompilerParams(needs_layout_passes=False)


@functools.partial(
    pl.kernel,
    out_type=jax.ShapeDtypeStruct((_NW, _PAD), jnp.float32),
    mesh=_mesh,
    compiler_params=_params,
    scratch_types=[
        pltpu.VMEM((_PAD,), jnp.float32),
        pltpu.VMEM((_N_SOURCE,), jnp.float32),
        pltpu.VMEM((_CHUNK,), jnp.int32),
        pltpu.VMEM((_CHUNK,), jnp.int32),
        pltpu.VMEM((_CHUNK,), jnp.int32),
        pltpu.VMEM((_CHUNK,), jnp.int32),
        pltpu.VMEM((_CHUNK,), jnp.float32),
        pltpu.VMEM((_CHUNK,), jnp.float32),
        pltpu.VMEM((_CHUNK,), jnp.float32),
        pltpu.VMEM((_CHUNK,), jnp.float32),
        pltpu.SemaphoreType.DMA,
        pltpu.SemaphoreType.DMA,
    ],
)
def _phase1(post_hbm, pre_hbm, w_hbm, wf_hbm, inp_hbm, part_hbm,
            acc, inp_v, post_a, post_b, pre_a, pre_b, w_a, w_b, wf_a, wf_b,
            sem_a, sem_b):
    cid = lax.axis_index("c")
    sid = lax.axis_index("s")
    wid = sid * _NC + cid
    tile_base = wid * _SYN_PER_TILE

    post_bufs = (post_a, post_b)
    pre_bufs = (pre_a, pre_b)
    w_bufs = (w_a, w_b)
    wf_bufs = (wf_a, wf_b)
    sems = (sem_a, sem_b)

    zeros16 = jnp.zeros((16,), jnp.float32)

    def zbody(i, _):
        acc[pl.ds(i * 16, 16)] = zeros16
        return 0

    lax.fori_loop(0, _PAD // 16, zbody, 0)

    pltpu.sync_copy(inp_hbm, inp_v)

    def issue_loads(b, chunk):
        syn0 = tile_base + chunk * _CHUNK
        sl = pl.ds(syn0, _CHUNK)
        pltpu.async_copy(post_hbm.at[sl], post_bufs[b], sems[b])
        pltpu.async_copy(pre_hbm.at[sl], pre_bufs[b], sems[b])
        pltpu.async_copy(w_hbm.at[sl], w_bufs[b], sems[b])
        pltpu.async_copy(wf_hbm.at[sl], wf_bufs[b], sems[b])

    def wait_loads(b):
        sl = pl.ds(0, _CHUNK)
        pltpu.make_async_copy(post_hbm.at[sl], post_bufs[b], sems[b]).wait()
        pltpu.make_async_copy(pre_hbm.at[sl], pre_bufs[b], sems[b]).wait()
        pltpu.make_async_copy(w_hbm.at[sl], w_bufs[b], sems[b]).wait()
        pltpu.make_async_copy(wf_hbm.at[sl], wf_bufs[b], sems[b]).wait()

    issue_loads(0, 0)
    issue_loads(1, 1)

    def compute_chunk(b):
        post_s, pre_s = post_bufs[b], pre_bufs[b]
        w_s, wf_s = w_bufs[b], wf_bufs[b]

        def cbody(i, _):
            posts, vals = [], []
            for u in range(5):
                sl = pl.ds(i * 80 + u * 16, 16)
                posts.append(post_s[sl])
                pre_v = pre_s[sl]
                act = plsc.load_gather(inp_v, [pre_v])
                vals.append(jnp.where(act > 0.0, w_s[sl] * wf_s[sl], 0.0))
            for u in range(5):
                plsc.addupdate_scatter(acc, [posts[u]], vals[u])
            return 0

        lax.fori_loop(0, _CHUNK // 80, cbody, 0)

    def tbody(t, _):
        for b in range(2):
            j = t * 2 + b
            wait_loads(b)
            compute_chunk(b)

            @pl.when(j + 2 < _NCHUNKS)
            def _():
                issue_loads(b, j + 2)
        return 0

    lax.fori_loop(0, _NCHUNKS // 2, tbody, 0)

    pltpu.sync_copy(acc, part_hbm.at[wid])


@functools.partial(
    pl.kernel,
    out_type=jax.ShapeDtypeStruct((_PAD,), jnp.float32),
    mesh=_mesh,
    compiler_params=_params,
    scratch_types=[
        pltpu.VMEM((_NW, _P2_COLS), jnp.float32),
        pltpu.VMEM((_P2_COLS,), jnp.float32),
    ],
)
def _phase2(part_hbm, out_hbm, buf, outv):
    cid = lax.axis_index("c")
    sid = lax.axis_index("s")
    wid = sid * _NC + cid
    off = wid * _P2_COLS

    pltpu.sync_copy(part_hbm.at[:, pl.ds(off, _P2_COLS)], buf)

    def rbody(i, _):
        s = pl.ds(i * 16, 16)
        v = buf[0, s]
        for k in range(1, _NW):
            v = v + buf[k, s]
        outv[s] = v
        return 0

    lax.fori_loop(0, _P2_COLS // 16, rbody, 0)

    pltpu.sync_copy(outv, out_hbm.at[pl.ds(off, _P2_COLS)])


def kernel(inputs_t, indices, weights, weights_factors):
    post = indices[:, 0]
    pre = indices[:, 1]
    inp_flat = jnp.reshape(inputs_t, (-1,))
    part = _phase1(post, pre, weights, weights_factors, inp_flat)
    out = _phase2(part)
    return out[: _N_TARGET].reshape(inputs_t.shape[0], -1)


# speedup vs baseline: 1.6300x; 1.2923x over previous
import functools

import jax
import jax.numpy as jnp
from jax import lax
from jax.experimental import pallas as pl
from jax.experimental.pallas import tpu as pltpu
from jax.experimental.pallas import tpu_sc as plsc

_N_TARGET = 100000
_N_SOURCE = 10000
_N_SYN = 6400000

_NC = 2
_NS = 16
_NW = _NC * _NS
_PAD = 102400
_SYN_PER_TILE = _N_SYN // _NW
_CHUNK = 2000
_NCHUNKS = _SYN_PER_TILE // _CHUNK
_P2_COLS = _PAD // _NW

_mesh = plsc.VectorSubcoreMesh(core_axis_name="c", subcore_axis_name="s")
_params = pltpu.CompilerParams(needs_layout_passes=False)


@functools.partial(
    pl.kernel,
    out_type=jax.ShapeDtypeStruct((_NW, _PAD), jnp.float32),
    mesh=_mesh,
    compiler_params=_params,
    scratch_types=[
        pltpu.VMEM((_PAD,), jnp.float32),
        pltpu.VMEM((_N_SOURCE,), jnp.float32),
        pltpu.VMEM((_CHUNK,), jnp.int32),
        pltpu.VMEM((_CHUNK,), jnp.int32),
        pltpu.VMEM((_CHUNK,), jnp.float32),
        pltpu.VMEM((_CHUNK,), jnp.float32),
        pltpu.VMEM((_CHUNK,), jnp.float32),
        pltpu.VMEM((_CHUNK,), jnp.float32),
        pltpu.SemaphoreType.DMA,
        pltpu.SemaphoreType.DMA,
    ],
)
def _phase1(pk_hbm, w_hbm, wf_hbm, inp_hbm, part_hbm,
            acc, inp_v, pk_a, pk_b, w_a, w_b, wf_a, wf_b,
            sem_a, sem_b):
    cid = lax.axis_index("c")
    sid = lax.axis_index("s")
    wid = sid * _NC + cid
    tile_base = wid * _SYN_PER_TILE

    pk_bufs = (pk_a, pk_b)
    w_bufs = (w_a, w_b)
    wf_bufs = (wf_a, wf_b)
    sems = (sem_a, sem_b)

    zeros16 = jnp.zeros((16,), jnp.float32)

    def zbody(i, _):
        for u in range(8):
            acc[pl.ds(i * 128 + u * 16, 16)] = zeros16
        return 0

    lax.fori_loop(0, _PAD // 128, zbody, 0)

    pltpu.sync_copy(inp_hbm, inp_v)

    def issue_loads(b, chunk):
        syn0 = tile_base + chunk * _CHUNK
        sl = pl.ds(syn0, _CHUNK)
        pltpu.async_copy(pk_hbm.at[sl], pk_bufs[b], sems[b])
        pltpu.async_copy(w_hbm.at[sl], w_bufs[b], sems[b])
        pltpu.async_copy(wf_hbm.at[sl], wf_bufs[b], sems[b])

    def wait_loads(b):
        sl = pl.ds(0, _CHUNK)
        pltpu.make_async_copy(pk_hbm.at[sl], pk_bufs[b], sems[b]).wait()
        pltpu.make_async_copy(w_hbm.at[sl], w_bufs[b], sems[b]).wait()
        pltpu.make_async_copy(wf_hbm.at[sl], wf_bufs[b], sems[b]).wait()

    issue_loads(0, 0)
    issue_loads(1, 1)

    def compute_chunk(b):
        pk_s, w_s, wf_s = pk_bufs[b], w_bufs[b], wf_bufs[b]

        def cbody(i, _):
            posts, vals = [], []
            for u in range(5):
                sl = pl.ds(i * 80 + u * 16, 16)
                pk_v = pk_s[sl]
                pre_v = jnp.bitwise_and(pk_v, 16383)
                posts.append(lax.shift_right_logical(pk_v, 14))
                act = plsc.load_gather(inp_v, [pre_v])
                vals.append(jnp.where(act > 0.0, w_s[sl] * wf_s[sl], 0.0))
            for u in range(5):
                plsc.addupdate_scatter(acc, [posts[u]], vals[u])
            return 0

        lax.fori_loop(0, _CHUNK // 80, cbody, 0)

    def tbody(t, _):
        for b in range(2):
            j = t * 2 + b
            wait_loads(b)
            compute_chunk(b)

            @pl.when(j + 2 < _NCHUNKS)
            def _():
                issue_loads(b, j + 2)
        return 0

    lax.fori_loop(0, _NCHUNKS // 2, tbody, 0)

    pltpu.sync_copy(acc, part_hbm.at[wid])


@functools.partial(
    pl.kernel,
    out_type=jax.ShapeDtypeStruct((_PAD,), jnp.float32),
    mesh=_mesh,
    compiler_params=_params,
    scratch_types=[
        pltpu.VMEM((_NW, _P2_COLS), jnp.float32),
        pltpu.VMEM((_P2_COLS,), jnp.float32),
    ],
)
def _phase2(part_hbm, out_hbm, buf, outv):
    cid = lax.axis_index("c")
    sid = lax.axis_index("s")
    wid = sid * _NC + cid
    off = wid * _P2_COLS

    pltpu.sync_copy(part_hbm.at[:, pl.ds(off, _P2_COLS)], buf)

    def rbody(i, _):
        s = pl.ds(i * 16, 16)
        v = buf[0, s]
        for k in range(1, _NW):
            v = v + buf[k, s]
        outv[s] = v
        return 0

    lax.fori_loop(0, _P2_COLS // 16, rbody, 0)

    pltpu.sync_copy(outv, out_hbm.at[pl.ds(off, _P2_COLS)])


def kernel(inputs_t, indices, weights, weights_factors):
    packed = jnp.bitwise_or(
        lax.shift_left(indices[:, 0], 14), indices[:, 1])
    inp_flat = jnp.reshape(inputs_t, (-1,))
    part = _phase1(packed, weights, weights_factors, inp_flat)
    out = _phase2(part)
    return out[: _N_TARGET].reshape(inputs_t.shape[0], -1)
